# baseline (speedup 1.0000x reference)
import functools

import jax
import jax.numpy as jnp
from jax import lax
from jax.experimental import pallas as pl
from jax.experimental.pallas import tpu as pltpu
from jax.experimental.pallas import tpu_sc as plsc

_KB = 3584


def _scores_argmax_body(x_ref, w_ref, age_ref, idx_ref, acc_ref, wsq_ref):
    k = pl.program_id(0)
    nk = pl.num_programs(0)

    @pl.when(k == 0)
    def _init():
        acc_ref[...] = jnp.zeros_like(acc_ref)
        wsq_ref[...] = jnp.zeros_like(wsq_ref)

    xb = x_ref[...]
    wb = w_ref[...]
    acc_ref[...] += lax.dot_general(
        xb, wb, (((1,), (1,)), ((), ())),
        preferred_element_type=jnp.float32,
        precision=lax.Precision.HIGHEST)
    ones = jnp.ones((1, wb.shape[1]), jnp.float32)
    wsq_ref[...] += lax.dot_general(
        ones, wb * wb, (((1,), (1,)), ((), ())),
        preferred_element_type=jnp.float32,
        precision=lax.Precision.HIGHEST)

    @pl.when(k == nk - 1)
    def _finish():
        scale = 1.0 / jnp.maximum(jnp.sqrt(wsq_ref[...]), 1e-12)
        act = (age_ref[...] >= 1.0).astype(jnp.float32)
        s = acc_ref[...] * (scale * act)
        m = jnp.max(s, axis=1, keepdims=True)
        ii = lax.broadcasted_iota(jnp.int32, s.shape, 1)
        cand = jnp.where(s == m, ii, jnp.int32(2**30))
        idx_ref[...] = jnp.min(cand, axis=1, keepdims=True)


def _scores_argmax(xf, W, age):
    B, K = xf.shape
    Y = W.shape[0]
    nk = K // _KB
    return pl.pallas_call(
        _scores_argmax_body,
        grid=(nk,),
        in_specs=[
            pl.BlockSpec((B, _KB), lambda k: (0, k)),
            pl.BlockSpec((Y, _KB), lambda k: (0, k)),
            pl.BlockSpec((1, Y), lambda k: (0, 0)),
        ],
        out_specs=pl.BlockSpec((B, 1), lambda k: (0, 0)),
        out_shape=jax.ShapeDtypeStruct((B, 1), jnp.int32),
        scratch_shapes=[
            pltpu.VMEM((B, Y), jnp.float32),
            pltpu.VMEM((1, Y), jnp.float32),
        ],
    )(xf, W, age)


def _sc_gather(table, idx):
    Yp, D = table.shape
    B = idx.shape[0]
    info = plsc.get_sparse_core_info()
    nw = info.num_cores * info.num_subcores
    bpw = B // nw
    mesh = plsc.VectorSubcoreMesh(core_axis_name="c", subcore_axis_name="s")

    @functools.partial(
        pl.kernel, mesh=mesh,
        out_type=jax.ShapeDtypeStruct((B, D), jnp.float32),
        scratch_types=[
            pltpu.VMEM((bpw,), jnp.int32),
            pltpu.VMEM((bpw, D), jnp.float32),
            pltpu.SemaphoreType.DMA,
        ],
    )
    def gk(table_hbm, idx_hbm, out_hbm, idx_v, rows_v, sem):
        wid = lax.axis_index("s") * info.num_cores + lax.axis_index("c")
        base = wid * bpw
        pltpu.sync_copy(idx_hbm.at[pl.ds(base, bpw)], idx_v)
        pltpu.async_copy(table_hbm.at[idx_v], rows_v, sem).wait()
        pltpu.sync_copy(rows_v, out_hbm.at[pl.ds(base, bpw)])

    return gk(table, idx)


def kernel(x, z, per_item, epo, x2, x3, x4, W_x2y, W_y2z, W_x2y4, y_neuron_age):
    B = x.shape[0]
    xf = x.reshape(B, -1)
    idx = _scores_argmax(xf, W_x2y, y_neuron_age)[:, 0]
    Z, Y = W_y2z.shape
    Dp = ((Z + 127) // 128) * 128
    table = jnp.zeros((Y, Dp), jnp.float32).at[:, :Z].set(W_y2z.T)
    out = _sc_gather(table, idx)
    return out[:, :Z]


# speedup vs baseline: 1.1469x; 1.1469x over previous
import functools

import jax
import jax.numpy as jnp
from jax import lax
from jax.experimental import pallas as pl
from jax.experimental.pallas import tpu as pltpu
from jax.experimental.pallas import tpu_sc as plsc

_RBX = 32
_RBW = 32


def _argmax_body(nx_steps, x_ref, w_ref, age_ref, idx_ref, xh_ref, gmax_ref, gidx_ref):
    k = pl.program_id(0)

    @pl.when(k < nx_steps)
    def _x_phase():
        xb = x_ref[...]
        n = jnp.sqrt(jnp.sum(xb * xb, axis=1, keepdims=True))
        xn = xb / jnp.maximum(n, 1e-12)
        xh_ref[pl.ds(k * _RBX, _RBX), :] = xn.astype(jnp.bfloat16)

    @pl.when(k >= nx_steps)
    def _w_phase():
        j = k - nx_steps
        wb = w_ref[...]
        n = jnp.sqrt(jnp.sum(wb * wb, axis=1, keepdims=True))
        wn = (wb / jnp.maximum(n, 1e-12)).astype(jnp.bfloat16)
        s = lax.dot_general(
            wn, xh_ref[...], (((1,), (1,)), ((), ())),
            preferred_element_type=jnp.float32)
        act = (age_ref[...] >= 1.0).astype(jnp.float32)
        s = s * act
        bm = jnp.max(s, axis=0, keepdims=True)
        ii = lax.broadcasted_iota(jnp.int32, s.shape, 0) + j * _RBW
        li = jnp.min(jnp.where(s == bm, ii, jnp.int32(2**30)),
                     axis=0, keepdims=True)

        @pl.when(j == 0)
        def _first():
            gmax_ref[...] = bm
            gidx_ref[...] = li

        @pl.when(j > 0)
        def _update():
            better = bm > gmax_ref[...]
            gidx_ref[...] = jnp.where(better, li, gidx_ref[...])
            gmax_ref[...] = jnp.maximum(bm, gmax_ref[...])

        @pl.when(k == pl.num_programs(0) - 1)
        def _emit():
            idx_ref[...] = gidx_ref[...]


def _scores_argmax(xf, W, age_col):
    B, K = xf.shape
    Y = W.shape[0]
    nx, nw = B // _RBX, Y // _RBW
    return pl.pallas_call(
        functools.partial(_argmax_body, nx),
        grid=(nx + nw,),
        in_specs=[
            pl.BlockSpec((_RBX, K), lambda k: (jnp.minimum(k, nx - 1), 0)),
            pl.BlockSpec((_RBW, K), lambda k: (jnp.maximum(k - nx, 0), 0)),
            pl.BlockSpec((_RBW, 1), lambda k: (jnp.maximum(k - nx, 0), 0)),
        ],
        out_specs=pl.BlockSpec((1, B), lambda k: (0, 0)),
        out_shape=jax.ShapeDtypeStruct((1, B), jnp.int32),
        scratch_shapes=[
            pltpu.VMEM((B, K), jnp.bfloat16),
            pltpu.VMEM((1, B), jnp.float32),
            pltpu.VMEM((1, B), jnp.int32),
        ],
    )(xf, W, age_col)


def _sc_gather(table, idx):
    Yp, D = table.shape
    B = idx.shape[0]
    info = plsc.get_sparse_core_info()
    nw = info.num_cores * info.num_subcores
    bpw = B // nw
    mesh = plsc.VectorSubcoreMesh(core_axis_name="c", subcore_axis_name="s")

    @functools.partial(
        pl.kernel, mesh=mesh,
        out_type=jax.ShapeDtypeStruct((B, D), jnp.float32),
        scratch_types=[
            pltpu.VMEM((bpw,), jnp.int32),
            pltpu.VMEM((bpw, D), jnp.float32),
            pltpu.SemaphoreType.DMA,
        ],
    )
    def gk(table_hbm, idx_hbm, out_hbm, idx_v, rows_v, sem):
        wid = lax.axis_index("s") * info.num_cores + lax.axis_index("c")
        base = wid * bpw
        pltpu.sync_copy(idx_hbm.at[pl.ds(base, bpw)], idx_v)
        pltpu.async_copy(table_hbm.at[idx_v], rows_v, sem).wait()
        pltpu.sync_copy(rows_v, out_hbm.at[pl.ds(base, bpw)])

    return gk(table, idx)


def kernel(x, z, per_item, epo, x2, x3, x4, W_x2y, W_y2z, W_x2y4, y_neuron_age):
    B = x.shape[0]
    xf = x.reshape(B, -1)
    age_col = y_neuron_age.reshape(-1, 1)
    idx = _scores_argmax(xf, W_x2y, age_col)[0]
    Z, Y = W_y2z.shape
    Dp = ((Z + 127) // 128) * 128
    table = jnp.zeros((Y, Dp), jnp.float32).at[:, :Z].set(W_y2z.T)
    out = _sc_gather(table, idx)
    return out[:, :Z]


# speedup vs baseline: 1.7060x; 1.4875x over previous
import functools

import jax
import jax.numpy as jnp
from jax import lax
from jax.experimental import pallas as pl
from jax.experimental.pallas import tpu as pltpu
from jax.experimental.pallas import tpu_sc as plsc

_RBX = 16
_RBW = 64


def _argmax_body(nx_steps, x_ref, w_ref, age_ref, idx_ref, xh_ref, gmax_ref, gidx_ref):
    k = pl.program_id(0)

    @pl.when(k < nx_steps)
    def _x_phase():
        xb = x_ref[...]
        n = jnp.sqrt(jnp.sum(xb * xb, axis=1, keepdims=True))
        inv = 1.0 / jnp.maximum(n, 1e-12)
        xh_ref[pl.ds(k * _RBX, _RBX), :] = (xb * inv).astype(jnp.bfloat16)

    @pl.when(k >= nx_steps)
    def _w_phase():
        j = k - nx_steps
        wb = w_ref[...]
        n = jnp.sqrt(jnp.sum(wb * wb, axis=1, keepdims=True))
        inv = 1.0 / jnp.maximum(n, 1e-12)
        wn = (wb * inv).astype(jnp.bfloat16)
        s = lax.dot_general(
            wn, xh_ref[...], (((1,), (1,)), ((), ())),
            preferred_element_type=jnp.float32)
        act = (age_ref[...] >= 1.0).astype(jnp.float32)
        s = s * act
        bm = jnp.max(s, axis=0, keepdims=True)
        ii = lax.broadcasted_iota(jnp.int32, s.shape, 0) + j * _RBW
        li = jnp.min(jnp.where(s == bm, ii, jnp.int32(2**30)),
                     axis=0, keepdims=True)

        @pl.when(j == 0)
        def _first():
            gmax_ref[...] = bm
            gidx_ref[...] = li

        @pl.when(j > 0)
        def _update():
            better = bm > gmax_ref[...]
            gidx_ref[...] = jnp.where(better, li, gidx_ref[...])
            gmax_ref[...] = jnp.maximum(bm, gmax_ref[...])

        @pl.when(k == pl.num_programs(0) - 1)
        def _emit():
            idx_ref[...] = gidx_ref[...]


def _scores_argmax(xf, W, age_col):
    B, K = xf.shape
    Y = W.shape[0]
    nx, nw = B // _RBX, Y // _RBW
    return pl.pallas_call(
        functools.partial(_argmax_body, nx),
        grid=(nx + nw,),
        in_specs=[
            pl.BlockSpec((_RBX, K), lambda k: (jnp.minimum(k, nx - 1), 0)),
            pl.BlockSpec((_RBW, K), lambda k: (jnp.maximum(k - nx, 0), 0)),
            pl.BlockSpec((_RBW, 1), lambda k: (jnp.maximum(k - nx, 0), 0)),
        ],
        out_specs=pl.BlockSpec((1, B), lambda k: (0, 0)),
        out_shape=jax.ShapeDtypeStruct((1, B), jnp.int32),
        scratch_shapes=[
            pltpu.VMEM((B, K), jnp.bfloat16),
            pltpu.VMEM((1, B), jnp.float32),
            pltpu.VMEM((1, B), jnp.int32),
        ],
    )(xf, W, age_col)


def _sc_gather(table, idx):
    Yp, D = table.shape
    B = idx.shape[0]
    info = plsc.get_sparse_core_info()
    nw = info.num_cores * info.num_subcores
    bpw = B // nw
    mesh = plsc.VectorSubcoreMesh(core_axis_name="c", subcore_axis_name="s")

    @functools.partial(
        pl.kernel, mesh=mesh,
        out_type=jax.ShapeDtypeStruct((B, D), jnp.float32),
        scratch_types=[
            pltpu.VMEM((bpw,), jnp.int32),
            pltpu.VMEM((bpw, D), jnp.float32),
            pltpu.SemaphoreType.DMA,
        ],
    )
    def gk(table_hbm, idx_hbm, out_hbm, idx_v, rows_v, sem):
        wid = lax.axis_index("s") * info.num_cores + lax.axis_index("c")
        base = wid * bpw
        pltpu.sync_copy(idx_hbm.at[pl.ds(base, bpw)], idx_v)
        pltpu.async_copy(table_hbm.at[idx_v], rows_v, sem).wait()
        pltpu.sync_copy(rows_v, out_hbm.at[pl.ds(base, bpw)])

    return gk(table, idx)


def kernel(x, z, per_item, epo, x2, x3, x4, W_x2y, W_y2z, W_x2y4, y_neuron_age):
    B = x.shape[0]
    xf = x.reshape(B, -1)
    age_col = y_neuron_age.reshape(-1, 1)
    idx = _scores_argmax(xf, W_x2y, age_col)[0]
    Z, Y = W_y2z.shape
    Dp = ((Z + 127) // 128) * 128
    table = jnp.zeros((Y, Dp), jnp.float32).at[:, :Z].set(W_y2z.T)
    out = _sc_gather(table, idx)
    return out[:, :Z]
